# Initial kernel scaffold; baseline (speedup 1.0000x reference)
import jax
import jax.numpy as jnp
from jax.experimental import pallas as pl


def kernel(z, codebooks):
    raise NotImplementedError("write your pallas kernel here")


# speedup vs baseline: 1.7039x; 1.7039x over previous
import jax
import jax.numpy as jnp
from jax.experimental import pallas as pl

NQ = 8
K = 1024
D = 256
TBLK = 512


def _split3(cb):
    bits = jax.lax.bitcast_convert_type(cb, jnp.uint32)
    p1 = jax.lax.bitcast_convert_type(bits & jnp.uint32(0xFFFF0000), jnp.float32)
    r1 = cb - p1
    bits1 = jax.lax.bitcast_convert_type(r1, jnp.uint32)
    p2 = jax.lax.bitcast_convert_type(bits1 & jnp.uint32(0xFFFF0000), jnp.float32)
    p3 = r1 - p2
    return p1.astype(jnp.bfloat16), p2.astype(jnp.bfloat16), p3.astype(jnp.bfloat16)


def _rvq_kernel(z_ref, cb_ref, cb2_ref, p1_ref, p2_ref, p3_ref,
                quant_ref, idx_ref, loss_ref, allq_ref):
    b = pl.program_id(0)
    t = pl.program_id(1)

    @pl.when(jnp.logical_and(b == 0, t == 0))
    def _init():
        loss_ref[...] = jnp.zeros_like(loss_ref)

    resid = z_ref[0]
    acc = jnp.zeros((TBLK, D), jnp.float32)
    iota = jax.lax.broadcasted_iota(jnp.int32, (TBLK, K), 1)

    for q in range(NQ):
        cb = cb_ref[q]
        dot = jax.lax.dot_general(
            resid, cb, (((1,), (1,)), ((), ())),
            preferred_element_type=jnp.float32)
        enc2 = jnp.sum(resid * resid, axis=1, keepdims=True)
        dist = (enc2 - 2.0 * dot) + cb2_ref[q][None, :]
        minv = jnp.min(dist, axis=1, keepdims=True)
        idx2 = jnp.min(jnp.where(dist == minv, iota, K), axis=1, keepdims=True)
        onehot = (iota == idx2).astype(jnp.bfloat16)
        g1 = jax.lax.dot_general(onehot, p1_ref[q], (((1,), (0,)), ((), ())),
                                 preferred_element_type=jnp.float32)
        g2 = jax.lax.dot_general(onehot, p2_ref[q], (((1,), (0,)), ((), ())),
                                 preferred_element_type=jnp.float32)
        g3 = jax.lax.dot_general(onehot, p3_ref[q], (((1,), (0,)), ((), ())),
                                 preferred_element_type=jnp.float32)
        z_q = (g1 + g2) + g3
        z_q_st = resid + (z_q - resid)
        acc = acc + z_q_st
        d = z_q - resid
        loss_ref[q, :] += jnp.sum(d * d, axis=0)
        idx_ref[0, q, :] = idx2[:, 0]
        allq_ref[q, 0, :, :] = z_q_st.T
        resid = resid - z_q_st

    quant_ref[0] = acc.T


@jax.jit
def kernel(z, codebooks):
    B, Dd, T = z.shape
    z_t = jnp.transpose(z, (0, 2, 1))
    cb2 = jnp.stack([jnp.sum(codebooks[i] ** 2, axis=1) for i in range(NQ)])
    p1, p2, p3 = _split3(codebooks)

    grid = (B, T // TBLK)
    quant, idx_t, loss_raw, all_q = pl.pallas_call(
        _rvq_kernel,
        grid=grid,
        in_specs=[
            pl.BlockSpec((1, TBLK, D), lambda b, t: (b, t, 0)),
            pl.BlockSpec((NQ, K, D), lambda b, t: (0, 0, 0)),
            pl.BlockSpec((NQ, K), lambda b, t: (0, 0)),
            pl.BlockSpec((NQ, K, D), lambda b, t: (0, 0, 0)),
            pl.BlockSpec((NQ, K, D), lambda b, t: (0, 0, 0)),
            pl.BlockSpec((NQ, K, D), lambda b, t: (0, 0, 0)),
        ],
        out_specs=[
            pl.BlockSpec((1, D, TBLK), lambda b, t: (b, 0, t)),
            pl.BlockSpec((1, NQ, TBLK), lambda b, t: (b, 0, t)),
            pl.BlockSpec((NQ, D), lambda b, t: (0, 0)),
            pl.BlockSpec((NQ, 1, D, TBLK), lambda b, t: (0, b, 0, t)),
        ],
        out_shape=[
            jax.ShapeDtypeStruct((B, Dd, T), jnp.float32),
            jax.ShapeDtypeStruct((B, NQ, T), jnp.int32),
            jax.ShapeDtypeStruct((NQ, D), jnp.float32),
            jax.ShapeDtypeStruct((NQ, B, Dd, T), jnp.float32),
        ],
    )(z_t, codebooks, cb2, p1, p2, p3)

    losses = jnp.sum(loss_raw, axis=1) * (1.0 / (B * Dd * T))
    all_idx = jnp.transpose(idx_t, (1, 0, 2))
    return quant, all_idx, losses, losses, all_q


# speedup vs baseline: 2.2379x; 1.3134x over previous
import jax
import jax.numpy as jnp
from jax.experimental import pallas as pl
from jax.experimental.pallas import tpu as pltpu

NQ = 8
K = 1024
D = 256
TBLK = 512
NCHAIN = 2


def _split3(cb):
    bits = jax.lax.bitcast_convert_type(cb, jnp.uint32)
    p1 = jax.lax.bitcast_convert_type(bits & jnp.uint32(0xFFFF0000), jnp.float32)
    r1 = cb - p1
    bits1 = jax.lax.bitcast_convert_type(r1, jnp.uint32)
    p2 = jax.lax.bitcast_convert_type(bits1 & jnp.uint32(0xFFFF0000), jnp.float32)
    p3 = r1 - p2
    return p1.astype(jnp.bfloat16), p2.astype(jnp.bfloat16), p3.astype(jnp.bfloat16)


def _rvq_kernel(z_ref, cb_ref, cb2_ref, p1_ref, p2_ref, p3_ref,
                quant_ref, idx_ref, loss_ref, allq_ref,
                resid_ref, acc_ref):
    b = pl.program_id(0)
    t = pl.program_id(1)

    @pl.when(jnp.logical_and(b == 0, t == 0))
    def _init():
        loss_ref[...] = jnp.zeros_like(loss_ref)

    iota = jax.lax.broadcasted_iota(jnp.int32, (TBLK, K), 1)
    resid_ref[...] = z_ref[0]
    acc_ref[...] = jnp.zeros_like(acc_ref)

    def step(q, _):
        cb = cb_ref[q]
        cb2row = cb2_ref[q][None, :]
        pieces = (p1_ref[q], p2_ref[q], p3_ref[q])
        for h in range(NCHAIN):
            sl = pl.ds(h * TBLK, TBLK)
            resid = resid_ref[sl, :]
            dot = jax.lax.dot_general(
                resid, cb, (((1,), (1,)), ((), ())),
                preferred_element_type=jnp.float32)
            enc2 = jnp.sum(resid * resid, axis=1, keepdims=True)
            dist = (enc2 - 2.0 * dot) + cb2row
            minv = jnp.min(dist, axis=1, keepdims=True)
            idx2 = jnp.min(jnp.where(dist == minv, iota, K), axis=1,
                           keepdims=True)
            onehot = (iota == idx2).astype(jnp.bfloat16)
            g1, g2, g3 = (jax.lax.dot_general(
                onehot, p, (((1,), (0,)), ((), ())),
                preferred_element_type=jnp.float32) for p in pieces)
            z_q = (g1 + g2) + g3
            z_q_st = resid + (z_q - resid)
            acc_ref[sl, :] += z_q_st
            d = z_q - resid
            loss_ref[q, :] += jnp.sum(d * d, axis=0)
            idx_ref[0, q, sl] = idx2[:, 0]
            allq_ref[q, 0, :, sl] = z_q_st.T
            resid_ref[sl, :] = resid - z_q_st
        return 0

    jax.lax.fori_loop(0, NQ, step, 0)
    quant_ref[0] = acc_ref[...].T


@jax.jit
def kernel(z, codebooks):
    B, Dd, T = z.shape
    z_t = jnp.transpose(z, (0, 2, 1))
    cb2 = jnp.stack([jnp.sum(codebooks[i] ** 2, axis=1) for i in range(NQ)])
    p1, p2, p3 = _split3(codebooks)

    cell = NCHAIN * TBLK
    grid = (B, T // cell)
    quant, idx_t, loss_raw, all_q = pl.pallas_call(
        _rvq_kernel,
        grid=grid,
        in_specs=[
            pl.BlockSpec((1, cell, D), lambda b, t: (b, t, 0)),
            pl.BlockSpec((NQ, K, D), lambda b, t: (0, 0, 0)),
            pl.BlockSpec((NQ, K), lambda b, t: (0, 0)),
            pl.BlockSpec((NQ, K, D), lambda b, t: (0, 0, 0)),
            pl.BlockSpec((NQ, K, D), lambda b, t: (0, 0, 0)),
            pl.BlockSpec((NQ, K, D), lambda b, t: (0, 0, 0)),
        ],
        out_specs=[
            pl.BlockSpec((1, D, cell), lambda b, t: (b, 0, t)),
            pl.BlockSpec((1, NQ, cell), lambda b, t: (b, 0, t)),
            pl.BlockSpec((NQ, D), lambda b, t: (0, 0)),
            pl.BlockSpec((NQ, 1, D, cell), lambda b, t: (0, b, 0, t)),
        ],
        out_shape=[
            jax.ShapeDtypeStruct((B, Dd, T), jnp.float32),
            jax.ShapeDtypeStruct((B, NQ, T), jnp.int32),
            jax.ShapeDtypeStruct((NQ, D), jnp.float32),
            jax.ShapeDtypeStruct((NQ, B, Dd, T), jnp.float32),
        ],
        scratch_shapes=[
            pltpu.VMEM((cell, D), jnp.float32),
            pltpu.VMEM((cell, D), jnp.float32),
        ],
    )(z_t, codebooks, cb2, p1, p2, p3)

    losses = jnp.sum(loss_raw, axis=1) * (1.0 / (B * Dd * T))
    all_idx = jnp.transpose(idx_t, (1, 0, 2))
    return quant, all_idx, losses, losses, all_q


# speedup vs baseline: 2.3141x; 1.0340x over previous
import jax
import jax.numpy as jnp
from jax.experimental import pallas as pl
from jax.experimental.pallas import tpu as pltpu

NQ = 8
K = 1024
D = 256
TBLK = 512
NCHAIN = 2


def _split3(cb):
    bits = jax.lax.bitcast_convert_type(cb, jnp.uint32)
    p1 = jax.lax.bitcast_convert_type(bits & jnp.uint32(0xFFFF0000), jnp.float32)
    r1 = cb - p1
    bits1 = jax.lax.bitcast_convert_type(r1, jnp.uint32)
    p2 = jax.lax.bitcast_convert_type(bits1 & jnp.uint32(0xFFFF0000), jnp.float32)
    p3 = r1 - p2
    return p1.astype(jnp.bfloat16), p2.astype(jnp.bfloat16), p3.astype(jnp.bfloat16)


def _rvq_kernel(z_ref, cb_ref, cb2_ref, p1_ref, p2_ref, p3_ref,
                quant_ref, idx_ref, loss_ref, allq_ref,
                resid_ref, acc_ref):
    b = pl.program_id(0)
    t = pl.program_id(1)

    @pl.when(jnp.logical_and(b == 0, t == 0))
    def _init():
        loss_ref[...] = jnp.zeros_like(loss_ref)

    iota = jax.lax.broadcasted_iota(jnp.int32, (TBLK, K), 1)
    resid_ref[...] = z_ref[0].T
    acc_ref[...] = jnp.zeros_like(acc_ref)

    def step(q, _):
        cb = cb_ref[q]
        cb2row = cb2_ref[q][None, :]
        pieces = (p1_ref[q], p2_ref[q], p3_ref[q])
        for h in range(NCHAIN):
            sl = pl.ds(h * TBLK, TBLK)
            resid = resid_ref[sl, :]
            dot = jax.lax.dot_general(
                resid, cb, (((1,), (1,)), ((), ())),
                preferred_element_type=jnp.float32)
            enc2 = jnp.sum(resid * resid, axis=1, keepdims=True)
            dist = (enc2 - 2.0 * dot) + cb2row
            minv = jnp.min(dist, axis=1, keepdims=True)
            idx2 = jnp.min(jnp.where(dist == minv, iota, K), axis=1,
                           keepdims=True)
            onehot = (iota == idx2).astype(jnp.bfloat16)
            g1, g2, g3 = (jax.lax.dot_general(
                onehot, p, (((1,), (0,)), ((), ())),
                preferred_element_type=jnp.float32) for p in pieces)
            z_q = (g1 + g2) + g3
            z_q_st = resid + (z_q - resid)
            acc_ref[sl, :] += z_q_st
            d = z_q - resid
            loss_ref[q, :] += jnp.sum(d * d, axis=0)
            idx_ref[0, q, sl] = idx2[:, 0]
            allq_ref[q, 0, :, sl] = z_q_st.T
            resid_ref[sl, :] = resid - z_q_st
        return 0

    jax.lax.fori_loop(0, NQ, step, 0)
    quant_ref[0] = acc_ref[...].T


@jax.jit
def kernel(z, codebooks):
    B, Dd, T = z.shape
    cb2 = jnp.stack([jnp.sum(codebooks[i] ** 2, axis=1) for i in range(NQ)])
    p1, p2, p3 = _split3(codebooks)

    cell = NCHAIN * TBLK
    grid = (B, T // cell)
    quant, idx_t, loss_raw, all_q = pl.pallas_call(
        _rvq_kernel,
        grid=grid,
        in_specs=[
            pl.BlockSpec((1, D, cell), lambda b, t: (b, 0, t)),
            pl.BlockSpec((NQ, K, D), lambda b, t: (0, 0, 0)),
            pl.BlockSpec((NQ, K), lambda b, t: (0, 0)),
            pl.BlockSpec((NQ, K, D), lambda b, t: (0, 0, 0)),
            pl.BlockSpec((NQ, K, D), lambda b, t: (0, 0, 0)),
            pl.BlockSpec((NQ, K, D), lambda b, t: (0, 0, 0)),
        ],
        out_specs=[
            pl.BlockSpec((1, D, cell), lambda b, t: (b, 0, t)),
            pl.BlockSpec((1, NQ, cell), lambda b, t: (b, 0, t)),
            pl.BlockSpec((NQ, D), lambda b, t: (0, 0)),
            pl.BlockSpec((NQ, 1, D, cell), lambda b, t: (0, b, 0, t)),
        ],
        out_shape=[
            jax.ShapeDtypeStruct((B, Dd, T), jnp.float32),
            jax.ShapeDtypeStruct((B, NQ, T), jnp.int32),
            jax.ShapeDtypeStruct((NQ, D), jnp.float32),
            jax.ShapeDtypeStruct((NQ, B, Dd, T), jnp.float32),
        ],
        scratch_shapes=[
            pltpu.VMEM((cell, D), jnp.float32),
            pltpu.VMEM((cell, D), jnp.float32),
        ],
    )(z, codebooks, cb2, p1, p2, p3)

    losses = jnp.sum(loss_raw, axis=1) * (1.0 / (B * Dd * T))
    all_idx = jnp.transpose(idx_t, (1, 0, 2))
    return quant, all_idx, losses, losses, all_q


# speedup vs baseline: 2.5091x; 1.0843x over previous
import jax
import jax.numpy as jnp
from jax.experimental import pallas as pl
from jax.experimental.pallas import tpu as pltpu

NQ = 8
K = 1024
D = 256
TBLK = 512
NCHAIN = 2


def _split3(cb):
    bits = jax.lax.bitcast_convert_type(cb, jnp.uint32)
    p1 = jax.lax.bitcast_convert_type(bits & jnp.uint32(0xFFFF0000), jnp.float32)
    r1 = cb - p1
    bits1 = jax.lax.bitcast_convert_type(r1, jnp.uint32)
    p2 = jax.lax.bitcast_convert_type(bits1 & jnp.uint32(0xFFFF0000), jnp.float32)
    p3 = r1 - p2
    return p1.astype(jnp.bfloat16), p2.astype(jnp.bfloat16), p3.astype(jnp.bfloat16)


def _rvq_kernel(z_ref, cb_ref, cb2_ref, p1_ref, p2_ref, p3_ref,
                quant_ref, idx_ref, loss_ref, allq_ref,
                resid_ref, acc_ref):
    b = pl.program_id(0)
    t = pl.program_id(1)

    @pl.when(jnp.logical_and(b == 0, t == 0))
    def _init():
        loss_ref[...] = jnp.zeros_like(loss_ref)

    iota = jax.lax.broadcasted_iota(jnp.int32, (TBLK, K), 1)
    resid_ref[...] = z_ref[0].T
    acc_ref[...] = jnp.zeros_like(acc_ref)

    def substep(q, h):
        sl = pl.ds(h * TBLK, TBLK)
        resid = resid_ref[sl, :]
        cb = cb_ref[q]
        dot2 = jax.lax.dot_general(
            resid + resid, cb, (((1,), (1,)), ((), ())),
            preferred_element_type=jnp.float32)
        enc2 = jnp.sum(resid * resid, axis=1, keepdims=True)
        dist = (enc2 - dot2) + cb2_ref[q][None, :]
        minv = jnp.min(dist, axis=1, keepdims=True)
        idx2 = jnp.min(jnp.where(dist == minv, iota, K), axis=1,
                       keepdims=True)
        onehot = (iota == idx2).astype(jnp.bfloat16)
        g1, g2, g3 = (jax.lax.dot_general(
            onehot, p, (((1,), (0,)), ((), ())),
            preferred_element_type=jnp.float32)
            for p in (p1_ref[q], p2_ref[q], p3_ref[q]))
        z_q = (g1 + g2) + g3
        z_q_st = resid + (z_q - resid)
        acc_ref[sl, :] += z_q_st
        d = z_q - resid
        loss_ref[q, :] += jnp.sum(d * d, axis=0)
        idx_ref[0, q, sl] = idx2[:, 0]
        allq_ref[q, 0, :, sl] = z_q_st.T
        resid_ref[sl, :] = resid - z_q_st

    def step(q2, _):
        for dq in range(2):
            for h in range(NCHAIN):
                substep(2 * q2 + dq, h)
        return 0

    jax.lax.fori_loop(0, NQ // 2, step, 0)
    quant_ref[0] = acc_ref[...].T


@jax.jit
def kernel(z, codebooks):
    B, Dd, T = z.shape
    cb2 = jnp.stack([jnp.sum(codebooks[i] ** 2, axis=1) for i in range(NQ)])
    p1, p2, p3 = _split3(codebooks)

    cell = NCHAIN * TBLK
    grid = (B, T // cell)
    quant, idx_t, loss_raw, all_q = pl.pallas_call(
        _rvq_kernel,
        grid=grid,
        in_specs=[
            pl.BlockSpec((1, D, cell), lambda b, t: (b, 0, t)),
            pl.BlockSpec((NQ, K, D), lambda b, t: (0, 0, 0)),
            pl.BlockSpec((NQ, K), lambda b, t: (0, 0)),
            pl.BlockSpec((NQ, K, D), lambda b, t: (0, 0, 0)),
            pl.BlockSpec((NQ, K, D), lambda b, t: (0, 0, 0)),
            pl.BlockSpec((NQ, K, D), lambda b, t: (0, 0, 0)),
        ],
        out_specs=[
            pl.BlockSpec((1, D, cell), lambda b, t: (b, 0, t)),
            pl.BlockSpec((1, NQ, cell), lambda b, t: (b, 0, t)),
            pl.BlockSpec((NQ, D), lambda b, t: (0, 0)),
            pl.BlockSpec((NQ, 1, D, cell), lambda b, t: (0, b, 0, t)),
        ],
        out_shape=[
            jax.ShapeDtypeStruct((B, Dd, T), jnp.float32),
            jax.ShapeDtypeStruct((B, NQ, T), jnp.int32),
            jax.ShapeDtypeStruct((NQ, D), jnp.float32),
            jax.ShapeDtypeStruct((NQ, B, Dd, T), jnp.float32),
        ],
        scratch_shapes=[
            pltpu.VMEM((cell, D), jnp.float32),
            pltpu.VMEM((cell, D), jnp.float32),
        ],
    )(z, codebooks, cb2, p1, p2, p3)

    losses = jnp.sum(loss_raw, axis=1) * (1.0 / (B * Dd * T))
    all_idx = jnp.transpose(idx_t, (1, 0, 2))
    return quant, all_idx, losses, losses, all_q


# speedup vs baseline: 2.5634x; 1.0216x over previous
import jax
import jax.numpy as jnp
from jax.experimental import pallas as pl
from jax.experimental.pallas import tpu as pltpu

NQ = 8
K = 1024
D = 256
TBLK = 512
NCHAIN = 2


def _split3(cb):
    bits = jax.lax.bitcast_convert_type(cb, jnp.uint32)
    p1 = jax.lax.bitcast_convert_type(bits & jnp.uint32(0xFFFF0000), jnp.float32)
    r1 = cb - p1
    bits1 = jax.lax.bitcast_convert_type(r1, jnp.uint32)
    p2 = jax.lax.bitcast_convert_type(bits1 & jnp.uint32(0xFFFF0000), jnp.float32)
    p3 = r1 - p2
    return p1.astype(jnp.bfloat16), p2.astype(jnp.bfloat16), p3.astype(jnp.bfloat16)


def _rvq_kernel(z_ref, cb_ref, cb2_ref, p1_ref, p2_ref, p3_ref,
                quant_ref, idx_ref, loss_ref, allq_ref,
                resid_ref, acc_ref):
    b = pl.program_id(0)
    t = pl.program_id(1)

    @pl.when(jnp.logical_and(b == 0, t == 0))
    def _init():
        loss_ref[...] = jnp.zeros_like(loss_ref)

    iota = jax.lax.broadcasted_iota(jnp.int32, (TBLK, K), 1)
    resid_ref[...] = z_ref[0].T
    acc_ref[...] = jnp.zeros_like(acc_ref)

    def substep(q, h):
        sl = pl.ds(h * TBLK, TBLK)
        resid = resid_ref[sl, :]
        cb = cb_ref[q]
        dot2 = jax.lax.dot_general(
            resid + resid, cb, (((1,), (1,)), ((), ())),
            preferred_element_type=jnp.float32)
        enc2 = jnp.sum(resid * resid, axis=1, keepdims=True)
        dist = (enc2 - dot2) + cb2_ref[q][None, :]
        minv = jnp.min(dist, axis=1, keepdims=True)
        idx2 = jnp.min(jnp.where(dist == minv, iota, K), axis=1,
                       keepdims=True)
        onehot = (iota == idx2).astype(jnp.bfloat16)
        g1, g2, g3 = (jax.lax.dot_general(
            onehot, p, (((1,), (0,)), ((), ())),
            preferred_element_type=jnp.float32)
            for p in (p1_ref[q], p2_ref[q], p3_ref[q]))
        z_q = (g1 + g2) + g3
        z_q_st = resid + (z_q - resid)
        acc_ref[sl, :] += z_q_st
        d = z_q - resid
        loss_ref[q, :] += jnp.sum(d * d, axis=0)
        idx_ref[0, q, sl] = idx2[:, 0]
        allq_ref[q, 0, :, sl] = z_q_st.T
        resid_ref[sl, :] = resid - z_q_st

    def step(q2, _):
        for dq in range(4):
            for h in range(NCHAIN):
                substep(4 * q2 + dq, h)
        return 0

    jax.lax.fori_loop(0, NQ // 4, step, 0)
    quant_ref[0] = acc_ref[...].T


@jax.jit
def kernel(z, codebooks):
    B, Dd, T = z.shape
    cb2 = jnp.stack([jnp.sum(codebooks[i] ** 2, axis=1) for i in range(NQ)])
    p1, p2, p3 = _split3(codebooks)

    cell = NCHAIN * TBLK
    grid = (B, T // cell)
    quant, idx_t, loss_raw, all_q = pl.pallas_call(
        _rvq_kernel,
        grid=grid,
        in_specs=[
            pl.BlockSpec((1, D, cell), lambda b, t: (b, 0, t)),
            pl.BlockSpec((NQ, K, D), lambda b, t: (0, 0, 0)),
            pl.BlockSpec((NQ, K), lambda b, t: (0, 0)),
            pl.BlockSpec((NQ, K, D), lambda b, t: (0, 0, 0)),
            pl.BlockSpec((NQ, K, D), lambda b, t: (0, 0, 0)),
            pl.BlockSpec((NQ, K, D), lambda b, t: (0, 0, 0)),
        ],
        out_specs=[
            pl.BlockSpec((1, D, cell), lambda b, t: (b, 0, t)),
            pl.BlockSpec((1, NQ, cell), lambda b, t: (b, 0, t)),
            pl.BlockSpec((NQ, D), lambda b, t: (0, 0)),
            pl.BlockSpec((NQ, 1, D, cell), lambda b, t: (0, b, 0, t)),
        ],
        out_shape=[
            jax.ShapeDtypeStruct((B, Dd, T), jnp.float32),
            jax.ShapeDtypeStruct((B, NQ, T), jnp.int32),
            jax.ShapeDtypeStruct((NQ, D), jnp.float32),
            jax.ShapeDtypeStruct((NQ, B, Dd, T), jnp.float32),
        ],
        scratch_shapes=[
            pltpu.VMEM((cell, D), jnp.float32),
            pltpu.VMEM((cell, D), jnp.float32),
        ],
    )(z, codebooks, cb2, p1, p2, p3)

    losses = jnp.sum(loss_raw, axis=1) * (1.0 / (B * Dd * T))
    all_idx = jnp.transpose(idx_t, (1, 0, 2))
    return quant, all_idx, losses, losses, all_q
